# baseline (speedup 1.0000x reference)
import functools

import jax
import jax.numpy as jnp
from jax import lax
from jax.experimental import pallas as pl
from jax.experimental.pallas import tpu as pltpu
from jax.experimental.pallas import tpu_sc as plsc

H = 64
SBF = 42

_NC = 2
_NS = 16
_NW = _NC * _NS

_CHUNK = 200


def _swish(x):
    return x * (1.0 / (1.0 + jnp.exp(-x)))


def _tc_pre_body(e_ref, Wcat_ref, g_ref):
    g_ref[...] = jnp.dot(e_ref[...], Wcat_ref[...],
                         preferred_element_type=jnp.float32)


def _make_tc_pre(E, BE):
    return pl.pallas_call(
        _tc_pre_body,
        grid=(E // BE,),
        in_specs=[
            pl.BlockSpec((BE, H), lambda i: (i, 0)),
            pl.BlockSpec((H, 2 * H), lambda i: (0, 0)),
        ],
        out_specs=pl.BlockSpec((BE, 2 * H), lambda i: (i, 0)),
        out_shape=jax.ShapeDtypeStruct((E, 2 * H), jnp.float32),
        compiler_params=pltpu.CompilerParams(
            dimension_semantics=("arbitrary",),
        ),
    )


def _make_sc_gather(T):
    tpw = T // _NW
    n_iter = tpw // _CHUNK
    mesh = plsc.VectorSubcoreMesh(core_axis_name="c", subcore_axis_name="s")

    @functools.partial(
        pl.kernel,
        mesh=mesh,
        out_type=jax.ShapeDtypeStruct((T, 2 * H), jnp.float32),
        scratch_types=[
            pltpu.VMEM((_CHUNK,), jnp.int32),
            pltpu.VMEM((_CHUNK,), jnp.int32),
            pltpu.VMEM((_CHUNK, 2 * H), jnp.float32),
            pltpu.VMEM((_CHUNK, 2 * H), jnp.float32),
            pltpu.SemaphoreType.DMA,
            pltpu.SemaphoreType.DMA,
        ],
    )
    def sc_gather(g_hbm, idx_ji_hbm, idx_kj_hbm, s_hbm,
                  idx1_v, idx2_v, buf1_v, buf2_v, sem1, sem2):
        wid = lax.axis_index("s") * _NC + lax.axis_index("c")

        def body(i, carry):
            base = pl.multiple_of(wid * tpw + i * _CHUNK, 8)
            pltpu.sync_copy(idx_ji_hbm.at[pl.ds(base, _CHUNK)], idx1_v)
            pltpu.sync_copy(idx_kj_hbm.at[pl.ds(base, _CHUNK)], idx2_v)
            cp1 = pltpu.async_copy(g_hbm.at[idx1_v], buf1_v, sem1)
            cp2 = pltpu.async_copy(g_hbm.at[idx2_v], buf2_v, sem2)
            cp1.wait()
            cp2.wait()

            def row(r, c):
                for gidx in range(H // 16):
                    a = buf1_v[r, pl.ds(gidx * 16, 16)]
                    b = buf2_v[r, pl.ds(H + gidx * 16, 16)]
                    buf1_v[r, pl.ds(gidx * 16, 16)] = a + b
                return c

            lax.fori_loop(0, _CHUNK, row, 0)
            pltpu.sync_copy(buf1_v, s_hbm.at[pl.ds(base, _CHUNK)])
            return carry

        lax.fori_loop(0, n_iter, body, 0)

    return sc_gather


def _tc_post_body(s_ref, area_ref, sbf_ref, Wsbf_ref, bsbf_ref,
                  W34_ref, blin_ref, w1_ref, b1_ref, out_ref):
    f32 = jnp.float32
    x = s_ref[:, 0:H]
    area_a = _swish(area_ref[...] * w1_ref[...] + b1_ref[...])
    x += jnp.dot(area_a, W34_ref[0:H], preferred_element_type=f32)
    sbf0 = _swish(
        jnp.dot(sbf_ref[...], Wsbf_ref[...], preferred_element_type=f32)
        + bsbf_ref[...]
    )
    x += jnp.dot(sbf0, W34_ref[H:2 * H], preferred_element_type=f32)
    out_ref[...] = _swish(x + blin_ref[...])


def _make_tc_post(T, BT):
    def row_blk(i):
        return (i, 0)

    def full_blk(i):
        return (0, 0)

    return pl.pallas_call(
        _tc_post_body,
        grid=(T // BT,),
        in_specs=[
            pl.BlockSpec((BT, 2 * H), row_blk),
            pl.BlockSpec((BT, H), row_blk),
            pl.BlockSpec((BT, SBF), row_blk),
            pl.BlockSpec((SBF, H), full_blk),
            pl.BlockSpec((1, H), full_blk),
            pl.BlockSpec((2 * H, H), full_blk),
            pl.BlockSpec((1, H), full_blk),
            pl.BlockSpec((1, 1), full_blk),
            pl.BlockSpec((1, 1), full_blk),
        ],
        out_specs=pl.BlockSpec((BT, H), row_blk),
        out_shape=jax.ShapeDtypeStruct((T, H), jnp.float32),
        compiler_params=pltpu.CompilerParams(
            dimension_semantics=("arbitrary",),
        ),
    )


def kernel(e, area, sbf, idx_ji, idx_kj, W_sbf, b_sbf, W_lin, b_lin,
           weight1, bias1):
    T = sbf.shape[0]
    E = e.shape[0]
    idx_ji = idx_ji.astype(jnp.int32)
    idx_kj = idx_kj.astype(jnp.int32)

    Wcat = jnp.concatenate([W_lin[0:H], W_lin[H:2 * H]], axis=1)

    g = _make_tc_pre(E, 2000)(e, Wcat)
    s = _make_sc_gather(T)(g, idx_ji, idx_kj)
    p = _make_tc_post(T, 2000)(
        s, area, sbf,
        W_sbf, b_sbf.reshape(1, H),
        W_lin[2 * H:4 * H], b_lin.reshape(1, H),
        weight1.reshape(1, 1), bias1.reshape(1, 1),
    )
    return p
